# baseline (speedup 1.0000x reference)
import jax
import jax.numpy as jnp
from jax.experimental import pallas as pl
from jax.experimental.pallas import tpu as pltpu

QK_DIM = 32
TOPK = 16
N = 2048
BATCH = 8
BQ = 256
LANES = 128
STACKS = 8
DEPTH = 2


def _topk_route_kernel(q_ref, k_ref, out_ref):
    scale = QK_DIM ** (-0.5)
    q = q_ref[0] * jnp.float32(scale)
    k = k_ref[0]
    logits = jax.lax.dot_general(
        q, k, (((1,), (1,)), ((), ())),
        preferred_element_type=jnp.float32)

    lane = jax.lax.broadcasted_iota(jnp.int32, (BQ, LANES), 1)
    neg = jnp.float32(-jnp.inf)

    hv, hi, mv, mi = [], [], [], []
    for g in range(STACKS):
        a = logits[:, (2 * g) * LANES:(2 * g + 1) * LANES]
        b = logits[:, (2 * g + 1) * LANES:(2 * g + 2) * LANES]
        ia = lane + (2 * g) * LANES
        ib = lane + (2 * g + 1) * LANES
        ge = a >= b
        hv.append(jnp.where(ge, a, b))
        hi.append(jnp.where(ge, ia, ib))
        mv.append(jnp.where(ge, b, a))
        mi.append(jnp.where(ge, ib, ia))

    outs = []
    for _ in range(TOPK):
        tvs, tis = list(hv), list(hi)
        while len(tvs) > 1:
            nvs, nis = [], []
            for p in range(0, len(tvs), 2):
                ge = tvs[p] >= tvs[p + 1]
                nvs.append(jnp.where(ge, tvs[p], tvs[p + 1]))
                nis.append(jnp.where(ge, tis[p], tis[p + 1]))
            tvs, tis = nvs, nis
        tv, ti = tvs[0], tis[0]
        wl = jnp.argmax(tv, axis=1, keepdims=True)
        widx = jnp.sum(jnp.where(lane == wl, ti, 0), axis=1,
                       keepdims=True)
        outs.append(widx)
        for g in range(STACKS):
            mask = hi[g] == widx
            hv[g] = jnp.where(mask, mv[g], hv[g])
            hi[g] = jnp.where(mask, mi[g], hi[g])
            mv[g] = jnp.where(mask, neg, mv[g])

    out_ref[0] = jnp.concatenate(outs, axis=1)


def kernel(query, key):
    grid = (BATCH, N // BQ)
    return pl.pallas_call(
        _topk_route_kernel,
        grid=grid,
        in_specs=[
            pl.BlockSpec((1, BQ, QK_DIM), lambda b, i: (b, i, 0)),
            pl.BlockSpec((1, N, QK_DIM), lambda b, i: (b, 0, 0)),
        ],
        out_specs=pl.BlockSpec((1, BQ, TOPK), lambda b, i: (b, i, 0)),
        out_shape=jax.ShapeDtypeStruct((BATCH, N, TOPK), jnp.int32),
        compiler_params=pltpu.CompilerParams(
            dimension_semantics=("parallel", "parallel")),
    )(query, key)
